# baseline (speedup 1.0000x reference)
import jax
import jax.numpy as jnp
from jax.experimental import pallas as pl
from jax.experimental.pallas import tpu as pltpu
from jax.experimental.pallas import tpu_sc as plsc

_WINDOW = 256


def _gather_rows(table2, idx_flat):
    n_idx = idx_flat.shape[0]
    d2 = table2.shape[1]
    mesh = plsc.VectorSubcoreMesh(core_axis_name="c", subcore_axis_name="s")

    @pl.kernel(
        out_type=jax.ShapeDtypeStruct((n_idx, d2), table2.dtype),
        mesh=mesh,
    )
    def sc_gather(table_hbm, idx_hbm, out_hbm):
        def body(idx_vmem, out_vmem):
            pltpu.sync_copy(table_hbm.at[idx_vmem], out_vmem)

        pltpu.emit_pipeline(
            body,
            grid=(n_idx // _WINDOW,),
            in_specs=[pl.BlockSpec((_WINDOW,), lambda i: (i,))],
            out_specs=[pl.BlockSpec((_WINDOW, d2), lambda i: (i, 0))],
            core_axis_name=("c", "s"),
            dimension_semantics=(pltpu.PARALLEL,),
        )(idx_hbm, out_hbm)

    return sc_gather(table2, idx_flat)


def kernel(x, table):
    b, s = x.shape
    v, d = table.shape
    idx = x.reshape(-1).astype(jnp.int32)
    rows2 = _gather_rows(table.reshape(v // 2, 2 * d), idx >> 1)
    lo = rows2[:, :d]
    hi = rows2[:, d:]
    out = jnp.where((idx & 1)[:, None] == 1, hi, lo)
    return out.reshape(b, s, d)


# speedup vs baseline: 1.4483x; 1.4483x over previous
import jax
import jax.numpy as jnp
from jax.experimental import pallas as pl
from jax.experimental.pallas import tpu as pltpu
from jax.experimental.pallas import tpu_sc as plsc

_PACK_BLK = 2048
_K_BLOCKS = 244
_K = _K_BLOCKS * _PACK_BLK
_NP_BLOCKS = _K_BLOCKS + 1
_NP = _NP_BLOCKS * _PACK_BLK
_WINDOW = 256


def _pack_table(table_t):
    d, v = table_t.shape

    def body(ta_ref, tb_ref, tout_ref):
        tout_ref[:, :d] = ta_ref[...].T
        tout_ref[:, d:] = tb_ref[...].T

    return pl.pallas_call(
        body,
        grid=(_NP_BLOCKS,),
        in_specs=[
            pl.BlockSpec((d, _PACK_BLK), lambda i: (0, i)),
            pl.BlockSpec((d, _PACK_BLK), lambda i: (0, i + _K_BLOCKS)),
        ],
        out_specs=pl.BlockSpec((_PACK_BLK, 2 * d), lambda i: (i, 0)),
        out_shape=jax.ShapeDtypeStruct((_NP, 2 * d), table_t.dtype),
        compiler_params=pltpu.CompilerParams(
            dimension_semantics=("parallel",)
        ),
    )(table_t, table_t)


def _gather_rows(packed, idx_flat):
    n_idx = idx_flat.shape[0]
    d2 = packed.shape[1]
    mesh = plsc.VectorSubcoreMesh(core_axis_name="c", subcore_axis_name="s")

    @pl.kernel(
        out_type=jax.ShapeDtypeStruct((n_idx, d2), packed.dtype),
        mesh=mesh,
    )
    def sc_gather(table_hbm, idx_hbm, out_hbm):
        def body(idx_vmem, out_vmem):
            pltpu.sync_copy(table_hbm.at[idx_vmem], out_vmem)

        pltpu.emit_pipeline(
            body,
            grid=(n_idx // _WINDOW,),
            in_specs=[pl.BlockSpec((_WINDOW,), lambda i: (i,))],
            out_specs=[pl.BlockSpec((_WINDOW, d2), lambda i: (i, 0))],
            core_axis_name=("c", "s"),
            dimension_semantics=(pltpu.PARALLEL,),
        )(idx_hbm, out_hbm)

    return sc_gather(packed, idx_flat)


def kernel(x, table):
    b, s = x.shape
    v, d = table.shape
    packed = _pack_table(table.T)
    idx = x.reshape(-1).astype(jnp.int32)
    hi_half = idx >= _K
    idx2 = jnp.where(hi_half, idx - _K, idx)
    rows2 = _gather_rows(packed, idx2)
    out = jnp.where(hi_half[:, None], rows2[:, d:], rows2[:, :d])
    return out.reshape(b, s, d)


# speedup vs baseline: 1.6408x; 1.1330x over previous
import jax
import jax.numpy as jnp
from jax.experimental import pallas as pl
from jax.experimental.pallas import tpu as pltpu
from jax.experimental.pallas import tpu_sc as plsc

_PACK_BLK = 4096
_K_BLOCKS = 122
_K = _K_BLOCKS * _PACK_BLK
_NP_BLOCKS = _K_BLOCKS + 1
_NP = _NP_BLOCKS * _PACK_BLK
_WINDOW = 256
_SEL_BLK = 2048


def _pack_table(table_t):
    d, v = table_t.shape

    def body(ta_ref, tb_ref, tout_ref):
        tout_ref[...] = jnp.concatenate(
            [ta_ref[...].T, tb_ref[...].T], axis=1
        )

    return pl.pallas_call(
        body,
        grid=(_NP_BLOCKS,),
        in_specs=[
            pl.BlockSpec((d, _PACK_BLK), lambda i: (0, i)),
            pl.BlockSpec((d, _PACK_BLK), lambda i: (0, i + _K_BLOCKS)),
        ],
        out_specs=pl.BlockSpec((_PACK_BLK, 2 * d), lambda i: (i, 0)),
        out_shape=jax.ShapeDtypeStruct((_NP, 2 * d), table_t.dtype),
        compiler_params=pltpu.CompilerParams(
            dimension_semantics=("parallel",)
        ),
    )(table_t, table_t)


def _gather_rows(packed, idx_flat):
    n_idx = idx_flat.shape[0]
    d2 = packed.shape[1]
    mesh = plsc.VectorSubcoreMesh(core_axis_name="c", subcore_axis_name="s")

    @pl.kernel(
        out_type=jax.ShapeDtypeStruct((n_idx, d2), packed.dtype),
        mesh=mesh,
    )
    def sc_gather(table_hbm, idx_hbm, out_hbm):
        def body(idx_vmem, out_vmem):
            pltpu.sync_copy(table_hbm.at[idx_vmem], out_vmem)

        pltpu.emit_pipeline(
            body,
            grid=(n_idx // _WINDOW,),
            in_specs=[pl.BlockSpec((_WINDOW,), lambda i: (i,))],
            out_specs=[pl.BlockSpec((_WINDOW, d2), lambda i: (i, 0))],
            core_axis_name=("c", "s"),
            dimension_semantics=(pltpu.PARALLEL,),
        )(idx_hbm, out_hbm)

    return sc_gather(packed, idx_flat)


def _select_half(rows2, hi_half, d):
    n = rows2.shape[0]

    def body(r_ref, h_ref, o_ref):
        r = r_ref[...]
        h = h_ref[...].reshape(_SEL_BLK, 1)
        o_ref[...] = jnp.where(h != 0, r[:, d:], r[:, :d])

    return pl.pallas_call(
        body,
        grid=(n // _SEL_BLK,),
        in_specs=[
            pl.BlockSpec((_SEL_BLK, 2 * d), lambda i: (i, 0)),
            pl.BlockSpec((_SEL_BLK,), lambda i: (i,)),
        ],
        out_specs=pl.BlockSpec((_SEL_BLK, d), lambda i: (i, 0)),
        out_shape=jax.ShapeDtypeStruct((n, d), rows2.dtype),
        compiler_params=pltpu.CompilerParams(
            dimension_semantics=("parallel",)
        ),
    )(rows2, hi_half)


def kernel(x, table):
    b, s = x.shape
    v, d = table.shape
    packed = _pack_table(table.T)
    idx = x.reshape(-1).astype(jnp.int32)
    hi_half = (idx >= _K).astype(jnp.int32)
    idx2 = jnp.where(hi_half != 0, idx - _K, idx)
    rows2 = _gather_rows(packed, idx2)
    out = _select_half(rows2, hi_half, d)
    return out.reshape(b, s, d)


# speedup vs baseline: 1.9507x; 1.1888x over previous
import jax
import jax.numpy as jnp
from jax.experimental import pallas as pl
from jax.experimental.pallas import tpu as pltpu
from jax.experimental.pallas import tpu_sc as plsc

_PACK_BLK = 4096
_K_BLOCKS = 122
_K = _K_BLOCKS * _PACK_BLK
_NP_BLOCKS = _K_BLOCKS + 1
_NP = _NP_BLOCKS * _PACK_BLK
_WINDOW = 256
_SEL_BLK = 2048


def _pack_table(table_t):
    d, v = table_t.shape

    def body(ta_ref, tb_ref, tout_ref):
        tout_ref[...] = jnp.concatenate(
            [ta_ref[...].T, tb_ref[...].T], axis=1
        )

    return pl.pallas_call(
        body,
        grid=(_NP_BLOCKS,),
        in_specs=[
            pl.BlockSpec((d, _PACK_BLK), lambda i: (0, i)),
            pl.BlockSpec((d, _PACK_BLK), lambda i: (0, i + _K_BLOCKS)),
        ],
        out_specs=pl.BlockSpec((_PACK_BLK, 2 * d), lambda i: (i, 0)),
        out_shape=jax.ShapeDtypeStruct((_NP, 2 * d), table_t.dtype),
        compiler_params=pltpu.CompilerParams(
            dimension_semantics=("parallel",)
        ),
    )(table_t, table_t)


def _gather_rows(table_rows, idx_flat):
    n_idx = idx_flat.shape[0]
    d = table_rows.shape[1]
    mesh = plsc.VectorSubcoreMesh(core_axis_name="c", subcore_axis_name="s")

    @pl.kernel(
        out_type=jax.ShapeDtypeStruct((n_idx, d), table_rows.dtype),
        mesh=mesh,
        compiler_params=pltpu.CompilerParams(use_tc_tiling_on_sc=False),
    )
    def sc_gather(table_hbm, idx_hbm, out_hbm):
        def body(idx_vmem, out_vmem):
            pltpu.sync_copy(table_hbm.at[idx_vmem], out_vmem)

        pltpu.emit_pipeline(
            body,
            grid=(n_idx // _WINDOW,),
            in_specs=[pl.BlockSpec((_WINDOW,), lambda i: (i,))],
            out_specs=[pl.BlockSpec((_WINDOW, d), lambda i: (i, 0))],
            core_axis_name=("c", "s"),
            dimension_semantics=(pltpu.PARALLEL,),
        )(idx_hbm, out_hbm)

    return sc_gather(table_rows, idx_flat)


def _select_half(rows2, hi_half, d):
    n = rows2.shape[0]

    def body(r_ref, h_ref, o_ref):
        r = r_ref[...]
        h = h_ref[...].reshape(_SEL_BLK, 1)
        o_ref[...] = jnp.where(h != 0, r[:, d:], r[:, :d])

    return pl.pallas_call(
        body,
        grid=(n // _SEL_BLK,),
        in_specs=[
            pl.BlockSpec((_SEL_BLK, 2 * d), lambda i: (i, 0)),
            pl.BlockSpec((_SEL_BLK,), lambda i: (i,)),
        ],
        out_specs=pl.BlockSpec((_SEL_BLK, d), lambda i: (i, 0)),
        out_shape=jax.ShapeDtypeStruct((n, d), rows2.dtype),
        compiler_params=pltpu.CompilerParams(
            dimension_semantics=("parallel",)
        ),
    )(rows2, hi_half)


def kernel(x, table):
    b, s = x.shape
    v, d = table.shape
    packed = _pack_table(table.T)
    idx = x.reshape(-1).astype(jnp.int32)
    idx2 = jnp.where(idx >= _K, 2 * (idx - _K) + 1, 2 * idx)
    out = _gather_rows(packed.reshape(2 * _NP, d), idx2)
    return out.reshape(b, s, d)


# speedup vs baseline: 2.2610x; 1.1591x over previous
import jax
import jax.numpy as jnp
from jax.experimental import pallas as pl
from jax.experimental.pallas import tpu as pltpu
from jax.experimental.pallas import tpu_sc as plsc

_PACK_BLK = 4096
_K_BLOCKS = 122
_K = _K_BLOCKS * _PACK_BLK
_NP_BLOCKS = _K_BLOCKS + 1
_NP = _NP_BLOCKS * _PACK_BLK
_WINDOW = 256
_SEL_BLK = 2048


def _pack_table(table_t):
    d, v = table_t.shape

    def body(ta_ref, tb_ref, tout_ref):
        tout_ref[...] = jnp.concatenate(
            [ta_ref[...], tb_ref[...]], axis=0
        ).T

    return pl.pallas_call(
        body,
        grid=(_NP_BLOCKS,),
        in_specs=[
            pl.BlockSpec((d, _PACK_BLK), lambda i: (0, i)),
            pl.BlockSpec((d, _PACK_BLK), lambda i: (0, i + _K_BLOCKS)),
        ],
        out_specs=pl.BlockSpec((_PACK_BLK, 2 * d), lambda i: (i, 0)),
        out_shape=jax.ShapeDtypeStruct((_NP, 2 * d), table_t.dtype),
        compiler_params=pltpu.CompilerParams(
            dimension_semantics=("parallel",)
        ),
    )(table_t, table_t)


def _gather_rows(table_rows, idx_flat):
    n_idx = idx_flat.shape[0]
    d = table_rows.shape[1]
    mesh = plsc.VectorSubcoreMesh(core_axis_name="c", subcore_axis_name="s")

    @pl.kernel(
        out_type=jax.ShapeDtypeStruct((n_idx, d), table_rows.dtype),
        mesh=mesh,
        compiler_params=pltpu.CompilerParams(use_tc_tiling_on_sc=False),
    )
    def sc_gather(table_hbm, idx_hbm, out_hbm):
        def body(idx_vmem, out_vmem):
            pltpu.sync_copy(table_hbm.at[idx_vmem], out_vmem)

        pltpu.emit_pipeline(
            body,
            grid=(n_idx // _WINDOW,),
            in_specs=[pl.BlockSpec((_WINDOW,), lambda i: (i,))],
            out_specs=[pl.BlockSpec((_WINDOW, d), lambda i: (i, 0))],
            core_axis_name=("c", "s"),
            dimension_semantics=(pltpu.PARALLEL,),
        )(idx_hbm, out_hbm)

    return sc_gather(table_rows, idx_flat)


def _select_half(rows2, hi_half, d):
    n = rows2.shape[0]

    def body(r_ref, h_ref, o_ref):
        r = r_ref[...]
        h = h_ref[...].reshape(_SEL_BLK, 1)
        o_ref[...] = jnp.where(h != 0, r[:, d:], r[:, :d])

    return pl.pallas_call(
        body,
        grid=(n // _SEL_BLK,),
        in_specs=[
            pl.BlockSpec((_SEL_BLK, 2 * d), lambda i: (i, 0)),
            pl.BlockSpec((_SEL_BLK,), lambda i: (i,)),
        ],
        out_specs=pl.BlockSpec((_SEL_BLK, d), lambda i: (i, 0)),
        out_shape=jax.ShapeDtypeStruct((n, d), rows2.dtype),
        compiler_params=pltpu.CompilerParams(
            dimension_semantics=("parallel",)
        ),
    )(rows2, hi_half)


def kernel(x, table):
    b, s = x.shape
    v, d = table.shape
    packed = _pack_table(table.T)
    idx = x.reshape(-1).astype(jnp.int32)
    idx2 = jnp.where(idx >= _K, 2 * (idx - _K) + 1, 2 * idx)
    out = _gather_rows(packed.reshape(2 * _NP, d), idx2)
    return out.reshape(b, s, d)


# speedup vs baseline: 2.4336x; 1.0763x over previous
import jax
import jax.numpy as jnp
from jax.experimental import pallas as pl
from jax.experimental.pallas import tpu as pltpu
from jax.experimental.pallas import tpu_sc as plsc

_PACK_BLK = 8192
_K_BLOCKS = 61
_K = _K_BLOCKS * _PACK_BLK
_NP_BLOCKS = _K_BLOCKS + 1
_NP = _NP_BLOCKS * _PACK_BLK
_WINDOW = 256
_SEL_BLK = 2048


def _pack_table(table_t):
    d, v = table_t.shape

    def body(ta_ref, tb_ref, tout_ref):
        tout_ref[...] = jnp.concatenate(
            [ta_ref[...], tb_ref[...]], axis=0
        ).T

    return pl.pallas_call(
        body,
        grid=(_NP_BLOCKS,),
        in_specs=[
            pl.BlockSpec((d, _PACK_BLK), lambda i: (0, i)),
            pl.BlockSpec((d, _PACK_BLK), lambda i: (0, i + _K_BLOCKS)),
        ],
        out_specs=pl.BlockSpec((_PACK_BLK, 2 * d), lambda i: (i, 0)),
        out_shape=jax.ShapeDtypeStruct((_NP, 2 * d), table_t.dtype),
        compiler_params=pltpu.CompilerParams(
            dimension_semantics=("parallel",)
        ),
    )(table_t, table_t)


def _gather_rows(table_rows, idx_flat):
    n_idx = idx_flat.shape[0]
    d = table_rows.shape[1]
    mesh = plsc.VectorSubcoreMesh(core_axis_name="c", subcore_axis_name="s")

    @pl.kernel(
        out_type=jax.ShapeDtypeStruct((n_idx, d), table_rows.dtype),
        mesh=mesh,
        compiler_params=pltpu.CompilerParams(use_tc_tiling_on_sc=False),
    )
    def sc_gather(table_hbm, idx_hbm, out_hbm):
        def body(idx_vmem, out_vmem):
            pltpu.sync_copy(table_hbm.at[idx_vmem], out_vmem)

        pltpu.emit_pipeline(
            body,
            grid=(n_idx // _WINDOW,),
            in_specs=[pl.BlockSpec((_WINDOW,), lambda i: (i,))],
            out_specs=[pl.BlockSpec((_WINDOW, d), lambda i: (i, 0))],
            core_axis_name=("c", "s"),
            dimension_semantics=(pltpu.PARALLEL,),
        )(idx_hbm, out_hbm)

    return sc_gather(table_rows, idx_flat)


def _select_half(rows2, hi_half, d):
    n = rows2.shape[0]

    def body(r_ref, h_ref, o_ref):
        r = r_ref[...]
        h = h_ref[...].reshape(_SEL_BLK, 1)
        o_ref[...] = jnp.where(h != 0, r[:, d:], r[:, :d])

    return pl.pallas_call(
        body,
        grid=(n // _SEL_BLK,),
        in_specs=[
            pl.BlockSpec((_SEL_BLK, 2 * d), lambda i: (i, 0)),
            pl.BlockSpec((_SEL_BLK,), lambda i: (i,)),
        ],
        out_specs=pl.BlockSpec((_SEL_BLK, d), lambda i: (i, 0)),
        out_shape=jax.ShapeDtypeStruct((n, d), rows2.dtype),
        compiler_params=pltpu.CompilerParams(
            dimension_semantics=("parallel",)
        ),
    )(rows2, hi_half)


def kernel(x, table):
    b, s = x.shape
    v, d = table.shape
    packed = _pack_table(table.T)
    idx = x.reshape(-1).astype(jnp.int32)
    idx2 = jnp.where(idx >= _K, 2 * (idx - _K) + 1, 2 * idx)
    out = _gather_rows(packed.reshape(2 * _NP, d), idx2)
    return out.reshape(b, s, d)


# speedup vs baseline: 2.4841x; 1.0207x over previous
import jax
import jax.numpy as jnp
from jax.experimental import pallas as pl
from jax.experimental.pallas import tpu as pltpu
from jax.experimental.pallas import tpu_sc as plsc

_PACK_BLK = 16384
_K_BLOCKS = 30
_K = _K_BLOCKS * _PACK_BLK
_NP_BLOCKS = _K_BLOCKS + 2
_NP = _NP_BLOCKS * _PACK_BLK
_WINDOW = 512
_SEL_BLK = 2048


def _pack_table(table_t):
    d, v = table_t.shape

    def body(ta_ref, tb_ref, tout_ref):
        tout_ref[...] = jnp.concatenate(
            [ta_ref[...], tb_ref[...]], axis=0
        ).T

    return pl.pallas_call(
        body,
        grid=(_NP_BLOCKS,),
        in_specs=[
            pl.BlockSpec((d, _PACK_BLK), lambda i: (0, i)),
            pl.BlockSpec((d, _PACK_BLK), lambda i: (0, i + _K_BLOCKS)),
        ],
        out_specs=pl.BlockSpec((_PACK_BLK, 2 * d), lambda i: (i, 0)),
        out_shape=jax.ShapeDtypeStruct((_NP, 2 * d), table_t.dtype),
        compiler_params=pltpu.CompilerParams(
            dimension_semantics=("parallel",)
        ),
    )(table_t, table_t)


def _gather_rows(table_rows, idx_flat):
    n_idx = idx_flat.shape[0]
    d = table_rows.shape[1]
    mesh = plsc.VectorSubcoreMesh(core_axis_name="c", subcore_axis_name="s")

    @pl.kernel(
        out_type=jax.ShapeDtypeStruct((n_idx, d), table_rows.dtype),
        mesh=mesh,
        compiler_params=pltpu.CompilerParams(use_tc_tiling_on_sc=False),
    )
    def sc_gather(table_hbm, idx_hbm, out_hbm):
        def body(idx_vmem, out_vmem):
            pltpu.sync_copy(table_hbm.at[idx_vmem], out_vmem)

        pltpu.emit_pipeline(
            body,
            grid=(n_idx // _WINDOW,),
            in_specs=[pl.BlockSpec((_WINDOW,), lambda i: (i,))],
            out_specs=[pl.BlockSpec((_WINDOW, d), lambda i: (i, 0))],
            core_axis_name=("c", "s"),
            dimension_semantics=(pltpu.PARALLEL,),
        )(idx_hbm, out_hbm)

    return sc_gather(table_rows, idx_flat)


def _select_half(rows2, hi_half, d):
    n = rows2.shape[0]

    def body(r_ref, h_ref, o_ref):
        r = r_ref[...]
        h = h_ref[...].reshape(_SEL_BLK, 1)
        o_ref[...] = jnp.where(h != 0, r[:, d:], r[:, :d])

    return pl.pallas_call(
        body,
        grid=(n // _SEL_BLK,),
        in_specs=[
            pl.BlockSpec((_SEL_BLK, 2 * d), lambda i: (i, 0)),
            pl.BlockSpec((_SEL_BLK,), lambda i: (i,)),
        ],
        out_specs=pl.BlockSpec((_SEL_BLK, d), lambda i: (i, 0)),
        out_shape=jax.ShapeDtypeStruct((n, d), rows2.dtype),
        compiler_params=pltpu.CompilerParams(
            dimension_semantics=("parallel",)
        ),
    )(rows2, hi_half)


def kernel(x, table):
    b, s = x.shape
    v, d = table.shape
    packed = _pack_table(table.T)
    idx = x.reshape(-1).astype(jnp.int32)
    idx2 = jnp.where(idx >= _K, 2 * (idx - _K) + 1, 2 * idx)
    out = _gather_rows(packed.reshape(2 * _NP, d), idx2)
    return out.reshape(b, s, d)
